# initial kernel scaffold (unmeasured)
import jax
import jax.numpy as jnp
from jax import lax
from jax.experimental import pallas as pl
from jax.experimental.pallas import tpu as pltpu

N_DEV = 4


def kernel(A, B):
    M, _ = A.shape
    _, N = B.shape
    MB = M // N_DEV

    def body(a_ref, b_ref, out_ref, comm_ref, send_sems, recv_sems):
        my = lax.axis_index("i")
        left = (my - 1) % N_DEV
        right = (my + 1) % N_DEV

        out_ref[...] = jnp.dot(
            a_ref[...], b_ref[...], preferred_element_type=jnp.float32
        )

        barrier_sem = pltpu.get_barrier_semaphore()

        def neighbor_barrier():
            for nbr in (left, right):
                pl.semaphore_signal(
                    barrier_sem,
                    inc=1,
                    device_id=(nbr,),
                    device_id_type=pl.DeviceIdType.MESH,
                )
            pl.semaphore_wait(barrier_sem, 2)

        neighbor_barrier()

        comm_ref[0, :, :] = out_ref[pl.ds(my * MB, MB), :]
        for s in range(N_DEV - 1):
            send_slot = s % 2
            recv_slot = (s + 1) % 2
            rdma = pltpu.make_async_remote_copy(
                src_ref=comm_ref.at[send_slot],
                dst_ref=comm_ref.at[recv_slot],
                send_sem=send_sems.at[s],
                recv_sem=recv_sems.at[s],
                device_id=(right,),
                device_id_type=pl.DeviceIdType.MESH,
            )
            rdma.start()
            rdma.wait()
            c = (my - 1 - s) % N_DEV
            comm_ref[recv_slot, :, :] = (
                comm_ref[recv_slot, :, :] + out_ref[pl.ds(c * MB, MB), :]
            )

        own = (my + 1) % N_DEV
        out_ref[pl.ds(own * MB, MB), :] = comm_ref[1, :, :]

        neighbor_barrier()

        for h in range(N_DEV - 1):
            send_slot = (1 + h) % 2
            recv_slot = h % 2
            rdma = pltpu.make_async_remote_copy(
                src_ref=comm_ref.at[send_slot],
                dst_ref=comm_ref.at[recv_slot],
                send_sem=send_sems.at[3 + h],
                recv_sem=recv_sems.at[3 + h],
                device_id=(right,),
                device_id_type=pl.DeviceIdType.MESH,
            )
            rdma.start()
            rdma.wait()
            c = (my - h) % N_DEV
            out_ref[pl.ds(c * MB, MB), :] = comm_ref[recv_slot, :, :]

    return pl.pallas_call(
        body,
        out_shape=jax.ShapeDtypeStruct((M, N), jnp.float32),
        in_specs=[
            pl.BlockSpec(memory_space=pltpu.VMEM),
            pl.BlockSpec(memory_space=pltpu.VMEM),
        ],
        out_specs=pl.BlockSpec(memory_space=pltpu.VMEM),
        scratch_shapes=[
            pltpu.VMEM((2, MB, N), jnp.float32),
            pltpu.SemaphoreType.DMA((2 * (N_DEV - 1),)),
            pltpu.SemaphoreType.DMA((2 * (N_DEV - 1),)),
        ],
        compiler_params=pltpu.CompilerParams(collective_id=0),
    )(A, B)


# baseline (device time: 706445 ns/iter reference)
import jax
import jax.numpy as jnp
from jax import lax
from jax.experimental import pallas as pl
from jax.experimental.pallas import tpu as pltpu

N_DEV = 4


def kernel(A, B):
    M, _ = A.shape
    _, N = B.shape
    MB = M // N_DEV

    def body(a_ref, b_ref, out_ref, comm_ref, send_sems, recv_sems, copy_sem):
        my = lax.axis_index("i")
        left = (my - 1) % N_DEV
        right = (my + 1) % N_DEV

        def partial_block(c):
            return jnp.dot(
                a_ref[pl.ds(c * MB, MB), :],
                b_ref[...],
                preferred_element_type=jnp.float32,
            )

        barrier_sem = pltpu.get_barrier_semaphore()

        def neighbor_barrier():
            for nbr in (left, right):
                pl.semaphore_signal(
                    barrier_sem,
                    inc=1,
                    device_id=(nbr,),
                    device_id_type=pl.DeviceIdType.MESH,
                )
            pl.semaphore_wait(barrier_sem, 2)

        comm_ref[0, :, :] = partial_block(my)
        neighbor_barrier()
        for s in range(N_DEV - 1):
            send_slot = s % 2
            recv_slot = (s + 1) % 2
            rdma = pltpu.make_async_remote_copy(
                src_ref=comm_ref.at[send_slot],
                dst_ref=comm_ref.at[recv_slot],
                send_sem=send_sems.at[s],
                recv_sem=recv_sems.at[s],
                device_id=(right,),
                device_id_type=pl.DeviceIdType.MESH,
            )
            rdma.start()
            rdma.wait()
            c = (my - 1 - s) % N_DEV
            comm_ref[recv_slot, :, :] = comm_ref[recv_slot, :, :] + partial_block(c)

        own = (my + 1) % N_DEV
        store_own = pltpu.make_async_copy(
            comm_ref.at[1], out_ref.at[pl.ds(own * MB, MB), :], copy_sem
        )
        store_own.start()
        store_own.wait()

        neighbor_barrier()

        for h in range(N_DEV - 1):
            send_slot = (1 + h) % 2
            recv_slot = h % 2
            rdma = pltpu.make_async_remote_copy(
                src_ref=comm_ref.at[send_slot],
                dst_ref=comm_ref.at[recv_slot],
                send_sem=send_sems.at[3 + h],
                recv_sem=recv_sems.at[3 + h],
                device_id=(right,),
                device_id_type=pl.DeviceIdType.MESH,
            )
            rdma.start()
            rdma.wait()
            c = (my - h) % N_DEV
            store = pltpu.make_async_copy(
                comm_ref.at[recv_slot], out_ref.at[pl.ds(c * MB, MB), :], copy_sem
            )
            store.start()
            store.wait()

    return pl.pallas_call(
        body,
        out_shape=jax.ShapeDtypeStruct((M, N), jnp.float32),
        in_specs=[
            pl.BlockSpec(memory_space=pltpu.VMEM),
            pl.BlockSpec(memory_space=pltpu.VMEM),
        ],
        out_specs=pl.BlockSpec(memory_space=pltpu.MemorySpace.HBM),
        scratch_shapes=[
            pltpu.VMEM((2, MB, N), jnp.float32),
            pltpu.SemaphoreType.DMA((2 * (N_DEV - 1),)),
            pltpu.SemaphoreType.DMA((2 * (N_DEV - 1),)),
            pltpu.SemaphoreType.DMA,
        ],
        compiler_params=pltpu.CompilerParams(
            collective_id=0, vmem_limit_bytes=100 * 1024 * 1024
        ),
    )(A, B)


# device time: 406344 ns/iter; 1.7385x vs baseline; 1.7385x over previous
import jax
import jax.numpy as jnp
from jax import lax
from jax.experimental import pallas as pl
from jax.experimental.pallas import tpu as pltpu

N_DEV = 4
R, L = 0, 1


def kernel(A, B):
    M, _ = A.shape
    _, N = B.shape
    MB = M // N_DEV
    HB = MB // 2

    def body(a_ref, b_ref, out_ref, comm_ref, tmp_ref, send_sems, recv_sems,
             copy_sems):
        my = lax.axis_index("i")
        left = (my - 1) % N_DEV
        right = (my + 1) % N_DEV

        def row0(c, ring):
            return c * MB + ring * HB

        def partial(c, ring):
            return jnp.dot(
                a_ref[pl.ds(row0(c, ring) , HB), :],
                b_ref[...],
                preferred_element_type=jnp.float32,
            )

        def ring_copy(ring, send_slot, recv_slot, hop):
            return pltpu.make_async_remote_copy(
                src_ref=comm_ref.at[ring, send_slot],
                dst_ref=comm_ref.at[ring, recv_slot],
                send_sem=send_sems.at[ring, hop],
                recv_sem=recv_sems.at[ring, hop],
                device_id=(right if ring == R else left,),
                device_id_type=pl.DeviceIdType.MESH,
            )

        barrier_sem = pltpu.get_barrier_semaphore()

        def neighbor_barrier():
            for nbr in (left, right):
                pl.semaphore_signal(
                    barrier_sem,
                    inc=1,
                    device_id=(nbr,),
                    device_id_type=pl.DeviceIdType.MESH,
                )
            pl.semaphore_wait(barrier_sem, 2)

        comm_ref[R, 0, :, :] = partial(my, R)
        comm_ref[L, 0, :, :] = partial(my, L)
        neighbor_barrier()
        tmp_ref[...] = partial((my - 1) % N_DEV, R)
        for s in range(N_DEV - 1):
            send_slot = s % 2
            recv_slot = (s + 1) % 2
            rdma_r = ring_copy(R, send_slot, recv_slot, s)
            rdma_l = ring_copy(L, send_slot, recv_slot, s)
            rdma_r.start()
            rdma_l.start()
            cr = (my - 1 - s) % N_DEV
            cl = (my + 1 + s) % N_DEV
            rdma_r.wait()
            comm_ref[R, recv_slot, :, :] = comm_ref[R, recv_slot, :, :] + tmp_ref[...]
            tmp_ref[...] = partial(cl, L)
            rdma_l.wait()
            comm_ref[L, recv_slot, :, :] = comm_ref[L, recv_slot, :, :] + tmp_ref[...]
            if s < N_DEV - 2:
                tmp_ref[...] = partial((my - 2 - s) % N_DEV, R)

        own_r = (my + 1) % N_DEV
        own_l = (my - 1) % N_DEV
        store_r = pltpu.make_async_copy(
            comm_ref.at[R, 1], out_ref.at[pl.ds(row0(own_r, R), HB), :],
            copy_sems.at[R],
        )
        store_l = pltpu.make_async_copy(
            comm_ref.at[L, 1], out_ref.at[pl.ds(row0(own_l, L), HB), :],
            copy_sems.at[L],
        )
        store_r.start()
        store_l.start()
        store_r.wait()
        store_l.wait()

        neighbor_barrier()

        for h in range(N_DEV - 1):
            send_slot = (1 + h) % 2
            recv_slot = h % 2
            rdma_r = ring_copy(R, send_slot, recv_slot, N_DEV - 1 + h)
            rdma_l = ring_copy(L, send_slot, recv_slot, N_DEV - 1 + h)
            rdma_r.start()
            rdma_l.start()
            cr = (my - h) % N_DEV
            cl = (my + h) % N_DEV
            rdma_r.wait()
            store_r = pltpu.make_async_copy(
                comm_ref.at[R, recv_slot], out_ref.at[pl.ds(row0(cr, R), HB), :],
                copy_sems.at[R],
            )
            store_r.start()
            rdma_l.wait()
            store_l = pltpu.make_async_copy(
                comm_ref.at[L, recv_slot], out_ref.at[pl.ds(row0(cl, L), HB), :],
                copy_sems.at[L],
            )
            store_l.start()
            store_r.wait()
            store_l.wait()

    return pl.pallas_call(
        body,
        out_shape=jax.ShapeDtypeStruct((M, N), jnp.float32),
        in_specs=[
            pl.BlockSpec(memory_space=pltpu.VMEM),
            pl.BlockSpec(memory_space=pltpu.VMEM),
        ],
        out_specs=pl.BlockSpec(memory_space=pltpu.MemorySpace.HBM),
        scratch_shapes=[
            pltpu.VMEM((2, 2, HB, N), jnp.float32),
            pltpu.VMEM((HB, N), jnp.float32),
            pltpu.SemaphoreType.DMA((2, 2 * (N_DEV - 1))),
            pltpu.SemaphoreType.DMA((2, 2 * (N_DEV - 1))),
            pltpu.SemaphoreType.DMA((2,)),
        ],
        compiler_params=pltpu.CompilerParams(
            collective_id=0, vmem_limit_bytes=100 * 1024 * 1024
        ),
    )(A, B)


# device time: 244638 ns/iter; 2.8877x vs baseline; 1.6610x over previous
import jax
import jax.numpy as jnp
from jax import lax
from jax.experimental import pallas as pl
from jax.experimental.pallas import tpu as pltpu

N_DEV = 4
R, L = 0, 1


def kernel(A, B):
    M, _ = A.shape
    _, N = B.shape
    MB = M // N_DEV
    HB = MB // 2

    def body(a_ref, b_ref, out_ref, comm_ref, tmp_ref, send_sems, recv_sems,
             copy_sems):
        my = lax.axis_index("i")
        left = (my - 1) % N_DEV
        right = (my + 1) % N_DEV

        def row0(c, ring):
            return c * MB + ring * HB

        def partial(c, ring):
            return jnp.dot(
                a_ref[pl.ds(row0(c, ring), HB), :],
                b_ref[...],
                preferred_element_type=jnp.float32,
            )

        def ring_copy(ring, send_slot, recv_slot, hop):
            return pltpu.make_async_remote_copy(
                src_ref=comm_ref.at[ring, send_slot],
                dst_ref=comm_ref.at[ring, recv_slot],
                send_sem=send_sems.at[ring, hop],
                recv_sem=recv_sems.at[ring, hop],
                device_id=(right if ring == R else left,),
                device_id_type=pl.DeviceIdType.MESH,
            )

        def accumulate(ring, slot):
            comm_ref[ring, slot, :, :] = (
                comm_ref[ring, slot, :, :].astype(jnp.float32)
                + tmp_ref[ring, :, :]
            ).astype(jnp.bfloat16)

        def store_from_tmp(ring, c):
            cp = pltpu.make_async_copy(
                tmp_ref.at[ring],
                out_ref.at[pl.ds(row0(c, ring), HB), :],
                copy_sems.at[ring],
            )
            cp.start()
            return cp

        barrier_sem = pltpu.get_barrier_semaphore()

        def neighbor_barrier():
            for nbr in (left, right):
                pl.semaphore_signal(
                    barrier_sem,
                    inc=1,
                    device_id=(nbr,),
                    device_id_type=pl.DeviceIdType.MESH,
                )
            pl.semaphore_wait(barrier_sem, 2)

        comm_ref[R, 0, :, :] = partial(my, R).astype(jnp.bfloat16)
        comm_ref[L, 0, :, :] = partial(my, L).astype(jnp.bfloat16)
        neighbor_barrier()
        tmp_ref[R, :, :] = partial((my - 1) % N_DEV, R)
        for s in range(N_DEV - 1):
            send_slot = s % 2
            recv_slot = (s + 1) % 2
            rdma_r = ring_copy(R, send_slot, recv_slot, s)
            rdma_l = ring_copy(L, send_slot, recv_slot, s)
            rdma_r.start()
            rdma_l.start()
            tmp_ref[L, :, :] = partial((my + 1 + s) % N_DEV, L)
            rdma_r.wait()
            accumulate(R, recv_slot)
            if s < N_DEV - 2:
                tmp_ref[R, :, :] = partial((my - 2 - s) % N_DEV, R)
            rdma_l.wait()
            accumulate(L, recv_slot)

        own_r = (my + 1) % N_DEV
        own_l = (my - 1) % N_DEV
        tmp_ref[R, :, :] = comm_ref[R, 1, :, :].astype(jnp.float32)
        tmp_ref[L, :, :] = comm_ref[L, 1, :, :].astype(jnp.float32)
        cp_r = store_from_tmp(R, own_r)
        cp_l = store_from_tmp(L, own_l)
        cp_r.wait()
        cp_l.wait()

        neighbor_barrier()

        for h in range(N_DEV - 1):
            send_slot = (1 + h) % 2
            recv_slot = h % 2
            rdma_r = ring_copy(R, send_slot, recv_slot, N_DEV - 1 + h)
            rdma_l = ring_copy(L, send_slot, recv_slot, N_DEV - 1 + h)
            rdma_r.start()
            rdma_l.start()
            rdma_r.wait()
            tmp_ref[R, :, :] = comm_ref[R, recv_slot, :, :].astype(jnp.float32)
            cp_r = store_from_tmp(R, (my - h) % N_DEV)
            rdma_l.wait()
            tmp_ref[L, :, :] = comm_ref[L, recv_slot, :, :].astype(jnp.float32)
            cp_l = store_from_tmp(L, (my + h) % N_DEV)
            cp_r.wait()
            cp_l.wait()

    return pl.pallas_call(
        body,
        out_shape=jax.ShapeDtypeStruct((M, N), jnp.float32),
        in_specs=[
            pl.BlockSpec(memory_space=pltpu.VMEM),
            pl.BlockSpec(memory_space=pltpu.VMEM),
        ],
        out_specs=pl.BlockSpec(memory_space=pltpu.MemorySpace.HBM),
        scratch_shapes=[
            pltpu.VMEM((2, 2, HB, N), jnp.bfloat16),
            pltpu.VMEM((2, HB, N), jnp.float32),
            pltpu.SemaphoreType.DMA((2, 2 * (N_DEV - 1))),
            pltpu.SemaphoreType.DMA((2, 2 * (N_DEV - 1))),
            pltpu.SemaphoreType.DMA((2,)),
        ],
        compiler_params=pltpu.CompilerParams(
            collective_id=0, vmem_limit_bytes=100 * 1024 * 1024
        ),
    )(A, B)


# device time: 222288 ns/iter; 3.1781x vs baseline; 1.1005x over previous
import jax
import jax.numpy as jnp
from jax import lax
from jax.experimental import pallas as pl
from jax.experimental.pallas import tpu as pltpu

N_DEV = 4
R, L = 0, 1


def kernel(A, B):
    M, _ = A.shape
    _, N = B.shape
    MB = M // N_DEV
    HB = MB // 2

    def body(a_ref, b_ref, out_ref, comm_ref, tmp_ref, send_sems, recv_sems,
             copy_sems):
        my = lax.axis_index("i")
        left = (my - 1) % N_DEV
        right = (my + 1) % N_DEV

        def row0(c, ring):
            return c * MB + ring * HB

        def partial(c, ring):
            return jnp.dot(
                a_ref[pl.ds(row0(c, ring), HB), :],
                b_ref[...],
                preferred_element_type=jnp.float32,
            )

        def ring_copy(ring, send_slot, recv_slot, hop):
            return pltpu.make_async_remote_copy(
                src_ref=comm_ref.at[ring, send_slot],
                dst_ref=comm_ref.at[ring, recv_slot],
                send_sem=send_sems.at[ring, hop],
                recv_sem=recv_sems.at[ring, hop],
                device_id=(right if ring == R else left,),
                device_id_type=pl.DeviceIdType.MESH,
            )

        def accumulate(ring, slot):
            comm_ref[ring, slot, :, :] = (
                comm_ref[ring, slot, :, :].astype(jnp.float32)
                + tmp_ref[ring, :, :]
            ).astype(jnp.bfloat16)

        def store_from_tmp(ring, c):
            cp = pltpu.make_async_copy(
                tmp_ref.at[ring],
                out_ref.at[pl.ds(row0(c, ring), HB), :],
                copy_sems.at[ring],
            )
            cp.start()
            return cp

        barrier_sem = pltpu.get_barrier_semaphore()

        def neighbor_barrier():
            for nbr in (left, right):
                pl.semaphore_signal(
                    barrier_sem,
                    inc=1,
                    device_id=(nbr,),
                    device_id_type=pl.DeviceIdType.MESH,
                )
            pl.semaphore_wait(barrier_sem, 2)

        comm_ref[R, 0, :, :] = partial(my, R).astype(jnp.bfloat16)
        comm_ref[L, 0, :, :] = partial(my, L).astype(jnp.bfloat16)
        neighbor_barrier()
        ring_copy(R, 0, 1, 0).start()
        ring_copy(L, 0, 1, 0).start()
        tmp_ref[R, :, :] = partial((my - 1) % N_DEV, R)
        tmp_ref[L, :, :] = partial((my + 1) % N_DEV, L)
        for s in range(N_DEV - 1):
            send_slot = s % 2
            recv_slot = (s + 1) % 2
            rdma_r = ring_copy(R, send_slot, recv_slot, s)
            rdma_l = ring_copy(L, send_slot, recv_slot, s)
            rdma_r.wait()
            accumulate(R, recv_slot)
            if s < N_DEV - 2:
                ring_copy(R, recv_slot, send_slot, s + 1).start()
            rdma_l.wait()
            accumulate(L, recv_slot)
            if s < N_DEV - 2:
                ring_copy(L, recv_slot, send_slot, s + 1).start()
                tmp_ref[R, :, :] = partial((my - 2 - s) % N_DEV, R)
                tmp_ref[L, :, :] = partial((my + 2 + s) % N_DEV, L)

        own_r = (my + 1) % N_DEV
        own_l = (my - 1) % N_DEV
        tmp_ref[R, :, :] = comm_ref[R, 1, :, :].astype(jnp.float32)
        tmp_ref[L, :, :] = comm_ref[L, 1, :, :].astype(jnp.float32)
        cp_r = store_from_tmp(R, own_r)
        cp_l = store_from_tmp(L, own_l)
        cp_r.wait()
        cp_l.wait()

        neighbor_barrier()

        ring_copy(R, 1, 0, N_DEV - 1).start()
        ring_copy(L, 1, 0, N_DEV - 1).start()
        for h in range(N_DEV - 1):
            send_slot = (1 + h) % 2
            recv_slot = h % 2
            rdma_r = ring_copy(R, send_slot, recv_slot, N_DEV - 1 + h)
            rdma_l = ring_copy(L, send_slot, recv_slot, N_DEV - 1 + h)
            rdma_r.wait()
            if h < N_DEV - 2:
                ring_copy(R, recv_slot, send_slot, N_DEV + h).start()
            tmp_ref[R, :, :] = comm_ref[R, recv_slot, :, :].astype(jnp.float32)
            cp_r = store_from_tmp(R, (my - h) % N_DEV)
            rdma_l.wait()
            if h < N_DEV - 2:
                ring_copy(L, recv_slot, send_slot, N_DEV + h).start()
            tmp_ref[L, :, :] = comm_ref[L, recv_slot, :, :].astype(jnp.float32)
            cp_l = store_from_tmp(L, (my + h) % N_DEV)
            cp_r.wait()
            cp_l.wait()

    return pl.pallas_call(
        body,
        out_shape=jax.ShapeDtypeStruct((M, N), jnp.float32),
        in_specs=[
            pl.BlockSpec(memory_space=pltpu.VMEM),
            pl.BlockSpec(memory_space=pltpu.VMEM),
        ],
        out_specs=pl.BlockSpec(memory_space=pltpu.MemorySpace.HBM),
        scratch_shapes=[
            pltpu.VMEM((2, 2, HB, N), jnp.bfloat16),
            pltpu.VMEM((2, HB, N), jnp.float32),
            pltpu.SemaphoreType.DMA((2, 2 * (N_DEV - 1))),
            pltpu.SemaphoreType.DMA((2, 2 * (N_DEV - 1))),
            pltpu.SemaphoreType.DMA((2,)),
        ],
        compiler_params=pltpu.CompilerParams(
            collective_id=0, vmem_limit_bytes=100 * 1024 * 1024
        ),
    )(A, B)
